# Initial kernel scaffold; baseline (speedup 1.0000x reference)
import jax
import jax.numpy as jnp
from jax.experimental import pallas as pl


def kernel(x, edge_index, W, b):
    raise NotImplementedError("write your pallas kernel here")


# speedup vs baseline: 15.2189x; 15.2189x over previous
import functools

import jax
import jax.numpy as jnp
from jax import lax
from jax.experimental import pallas as pl
from jax.experimental.pallas import tpu as pltpu
from jax.experimental.pallas import tpu_sc as plsc

NC = 2
NS = 16
NW = NC * NS
B = 128
LANES = 16
BN = 512


def _round_up(v, m):
    return (v + m - 1) // m * m


def _deg_body(np_, ep, cols, dacc_out, idx_c, ones_b, zero_b, acc):
    c = lax.axis_index("c")
    s = lax.axis_index("s")
    one = jnp.full((LANES,), 1.0, jnp.float32)
    zero = jnp.zeros((LANES,), jnp.float32)

    @pl.loop(0, B)
    def _init(r):
        ones_b[r, :] = one
        zero_b[r, :] = zero

    rpt = np_ // NS
    for k in range(rpt // B):
        pltpu.sync_copy(zero_b, acc.at[pl.ds(s * rpt + k * B, B)])
    plsc.subcore_barrier()

    ew = ep // NW
    base = (c * NS + s) * ew

    @pl.loop(0, ew // B)
    def _edges(blk):
        off = pl.multiple_of(base + blk * B, B)
        pltpu.sync_copy(cols.at[pl.ds(off, B)], idx_c)
        pltpu.sync_copy(ones_b, acc.at[idx_c], add=True)

    plsc.subcore_barrier()
    pltpu.sync_copy(acc.at[pl.ds(s * rpt, rpt)],
                    dacc_out.at[c, pl.ds(s * rpt, rpt), :])


def _hop_body(np_, ep, d, g, rows, cols, part_out, idx_r, idx_c, rowbuf, acc):
    c = lax.axis_index("c")
    s = lax.axis_index("s")
    zero = jnp.zeros((LANES,), jnp.float32)

    @pl.loop(0, B)
    def _zero(r):
        for j in range(d // LANES):
            rowbuf[r, pl.ds(j * LANES, LANES)] = zero

    rpt = np_ // NS
    for k in range(rpt // B):
        pltpu.sync_copy(rowbuf, acc.at[pl.ds(s * rpt + k * B, B)])
    plsc.subcore_barrier()

    ew = ep // NW
    base = (c * NS + s) * ew

    @pl.loop(0, ew // B)
    def _edges(blk):
        off = pl.multiple_of(base + blk * B, B)
        pltpu.sync_copy(rows.at[pl.ds(off, B)], idx_r)
        pltpu.sync_copy(cols.at[pl.ds(off, B)], idx_c)
        pltpu.sync_copy(g.at[idx_r], rowbuf)
        pltpu.sync_copy(rowbuf, acc.at[idx_c], add=True)

    plsc.subcore_barrier()
    pltpu.sync_copy(acc.at[pl.ds(s * rpt, rpt)],
                    part_out.at[c, pl.ds(s * rpt, rpt), :])


@functools.lru_cache(maxsize=None)
def _deg_call(np_, ep):
    mesh = plsc.VectorSubcoreMesh(core_axis_name="c", subcore_axis_name="s")
    return pl.kernel(
        functools.partial(_deg_body, np_, ep),
        out_type=jax.ShapeDtypeStruct((NC, np_, LANES), jnp.float32),
        mesh=mesh,
        scratch_types=[
            pltpu.VMEM((B,), jnp.int32),
            pltpu.VMEM((B, LANES), jnp.float32),
            pltpu.VMEM((B, LANES), jnp.float32),
            pltpu.VMEM_SHARED((np_, LANES), jnp.float32),
        ],
    )


@functools.lru_cache(maxsize=None)
def _hop_call(np_, ep, d):
    mesh = plsc.VectorSubcoreMesh(core_axis_name="c", subcore_axis_name="s")
    return pl.kernel(
        functools.partial(_hop_body, np_, ep, d),
        out_type=jax.ShapeDtypeStruct((NC, np_, d), jnp.float32),
        mesh=mesh,
        scratch_types=[
            pltpu.VMEM((B,), jnp.int32),
            pltpu.VMEM((B,), jnp.int32),
            pltpu.VMEM((B, d), jnp.float32),
            pltpu.VMEM_SHARED((np_, d), jnp.float32),
        ],
    )


def _prep_body(d0, d1, x, g0, disb):
    deg = d0[:, 0:1] + d1[:, 0:1] + 1.0
    dis = lax.rsqrt(deg)
    g0[...] = dis * x[...]
    disb[...] = jnp.broadcast_to(dis, disb.shape)


def _comb_body(p0, p1, g0, db, g1):
    d = db[...]
    g1[...] = d * d * (p0[...] + p1[...] + g0[...])


def _final_body(q0, q1, g1, db, w, b, o):
    h = db[...] * (q0[...] + q1[...] + g1[...])
    o[...] = lax.dot_general(h, w[...], (((1,), (1,)), ((), ())),
                             preferred_element_type=jnp.float32) + b[...]


def _prep(d0, d1, xp):
    np_, d = xp.shape
    grid = np_ // BN
    bs = lambda shp: pl.BlockSpec(shp, lambda i: (i, 0))
    return pl.pallas_call(
        _prep_body,
        grid=(grid,),
        in_specs=[bs((BN, LANES)), bs((BN, LANES)), bs((BN, d))],
        out_specs=[bs((BN, d)), bs((BN, d))],
        out_shape=[jax.ShapeDtypeStruct((np_, d), jnp.float32)] * 2,
    )(d0, d1, xp)


def _comb(p0, p1, g0, disb):
    np_, d = g0.shape
    grid = np_ // BN
    bs = pl.BlockSpec((BN, d), lambda i: (i, 0))
    return pl.pallas_call(
        _comb_body,
        grid=(grid,),
        in_specs=[bs] * 4,
        out_specs=bs,
        out_shape=jax.ShapeDtypeStruct((np_, d), jnp.float32),
    )(p0, p1, g0, disb)


def _final(q0, q1, g1, disb, w, b2):
    np_, d = g1.shape
    grid = np_ // BN
    bs = pl.BlockSpec((BN, d), lambda i: (i, 0))
    return pl.pallas_call(
        _final_body,
        grid=(grid,),
        in_specs=[bs, bs, bs, bs,
                  pl.BlockSpec((d, d), lambda i: (0, 0)),
                  pl.BlockSpec((1, d), lambda i: (0, 0))],
        out_specs=bs,
        out_shape=jax.ShapeDtypeStruct((np_, d), jnp.float32),
    )(q0, q1, g1, disb, w, b2)


def kernel(x, edge_index, W, b):
    n, d = x.shape
    e = edge_index.shape[1]
    np_ = _round_up(n, NS * B)
    ep = _round_up(e, NW * B)

    rows = edge_index[0].astype(jnp.int32)
    cols = edge_index[1].astype(jnp.int32)
    pad = ep - e
    ar = jnp.arange(pad, dtype=jnp.int32)
    rows_p = jnp.concatenate([rows, ar % 64])
    cols_p = jnp.concatenate([cols, n + ar % (np_ - n)])
    xp = jnp.pad(x, ((0, np_ - n), (0, 0)))

    dacc = _deg_call(np_, ep)(cols_p)
    g0, disb = _prep(dacc[0], dacc[1], xp)
    p = _hop_call(np_, ep, d)(g0, rows_p, cols_p)
    g1 = _comb(p[0], p[1], g0, disb)
    q = _hop_call(np_, ep, d)(g1, rows_p, cols_p)
    out = _final(q[0], q[1], g1, disb, W, b.reshape(1, d))
    return out[:n]


# speedup vs baseline: 21.9644x; 1.4432x over previous
import functools

import jax
import jax.numpy as jnp
from jax import lax
from jax.experimental import pallas as pl
from jax.experimental.pallas import tpu as pltpu
from jax.experimental.pallas import tpu_sc as plsc

NC = 2
NS = 16
NW = NC * NS
B = 128
LANES = 16
TC_GRID = 16
DEG_K = 8


def _round_up(v, m):
    return (v + m - 1) // m * m


def _deg_body(np_, nblk, d, cols, zh, oh, dacc_out, ia, ones_b, acc):
    c = lax.axis_index("c")
    s = lax.axis_index("s")
    pltpu.sync_copy(oh, ones_b)

    rpt = np_ // NS
    pltpu.sync_copy(zh.at[pl.ds(s * rpt, rpt)], acc.at[pl.ds(s * rpt, rpt)])
    plsc.subcore_barrier()

    w = c * NS + s

    @pl.loop(0, nblk)
    def _edges(blk):
        pltpu.sync_copy(cols.at[w, blk], ia)
        pltpu.sync_copy(ones_b, acc.at[ia], add=True)

    plsc.subcore_barrier()
    pltpu.sync_copy(acc.at[pl.ds(s * rpt, rpt)],
                    dacc_out.at[c, pl.ds(s * rpt, rpt), :])


def _hop_body(np_, nblk, d, g, rows, cols, zh, part_out,
              idx_r, idx_c, buf0, buf1, sem0, sem1, semir, semic, acc):
    c = lax.axis_index("c")
    s = lax.axis_index("s")

    rpt = np_ // NS
    pltpu.sync_copy(zh.at[pl.ds(s * rpt, rpt)], acc.at[pl.ds(s * rpt, rpt)])
    plsc.subcore_barrier()

    w = c * NS + s
    pltpu.sync_copy(rows.at[w, pl.ds(0, 2)], idx_r.at[0])
    pltpu.sync_copy(cols.at[w, pl.ds(0, 2)], idx_c.at[0])
    pltpu.async_copy(g.at[idx_r.at[0, 0]], buf0, sem0)
    pltpu.async_copy(rows.at[w, pl.ds(2, 2)], idx_r.at[1], semir)
    pltpu.async_copy(cols.at[w, pl.ds(2, 2)], idx_c.at[1], semic)

    def _pair(pr, p, q):
        pltpu.make_async_copy(g.at[idx_r.at[p, 0]], buf0, sem0).wait()
        pltpu.async_copy(g.at[idx_r.at[p, 1]], buf1, sem1)
        pltpu.sync_copy(buf0, acc.at[idx_c.at[p, 0]], add=True)
        pltpu.make_async_copy(g.at[idx_r.at[p, 1]], buf1, sem1).wait()
        pltpu.make_async_copy(rows.at[w, pl.ds(0, 2)], idx_r.at[q],
                              semir).wait()
        pltpu.make_async_copy(cols.at[w, pl.ds(0, 2)], idx_c.at[q],
                              semic).wait()
        pltpu.async_copy(g.at[idx_r.at[q, 0]], buf0, sem0)
        pltpu.sync_copy(buf1, acc.at[idx_c.at[p, 1]], add=True)
        nx = lax.rem(2 * pr + 4, nblk)
        pltpu.async_copy(rows.at[w, pl.ds(nx, 2)], idx_r.at[p], semir)
        pltpu.async_copy(cols.at[w, pl.ds(nx, 2)], idx_c.at[p], semic)

    @pl.loop(0, nblk // 2, step=2)
    def _edges(pr):
        _pair(pr, 0, 1)
        _pair(pr + 1, 1, 0)

    pltpu.make_async_copy(g.at[idx_r.at[0, 0]], buf0, sem0).wait()
    pltpu.make_async_copy(rows.at[w, pl.ds(0, 2)], idx_r.at[0], semir).wait()
    pltpu.make_async_copy(cols.at[w, pl.ds(0, 2)], idx_c.at[0], semic).wait()
    plsc.subcore_barrier()
    pltpu.sync_copy(acc.at[pl.ds(s * rpt, rpt)],
                    part_out.at[c, pl.ds(s * rpt, rpt), :])


@functools.lru_cache(maxsize=None)
def _deg_call(np_, nblk, d):
    mesh = plsc.VectorSubcoreMesh(core_axis_name="c", subcore_axis_name="s")
    return pl.kernel(
        functools.partial(_deg_body, np_, nblk, d),
        out_type=jax.ShapeDtypeStruct((NC, np_, d), jnp.float32),
        mesh=mesh,
        scratch_types=[
            pltpu.VMEM((B,), jnp.int32),
            pltpu.VMEM((B, d), jnp.float32),
            pltpu.VMEM_SHARED((np_, d), jnp.float32),
        ],
    )


@functools.lru_cache(maxsize=None)
def _hop_call(np_, nblk, d):
    mesh = plsc.VectorSubcoreMesh(core_axis_name="c", subcore_axis_name="s")
    return pl.kernel(
        functools.partial(_hop_body, np_, nblk, d),
        out_type=jax.ShapeDtypeStruct((NC, np_, d), jnp.float32),
        mesh=mesh,
        scratch_types=[
            pltpu.VMEM((2, 2, B), jnp.int32),
            pltpu.VMEM((2, 2, B), jnp.int32),
            pltpu.VMEM((B, d), jnp.float32),
            pltpu.VMEM((B, d), jnp.float32),
            pltpu.SemaphoreType.DMA,
            pltpu.SemaphoreType.DMA,
            pltpu.SemaphoreType.DMA,
            pltpu.SemaphoreType.DMA,
            pltpu.VMEM_SHARED((np_, d), jnp.float32),
        ],
    )


def _prep_body(dacc, x, g0, disb):
    deg = dacc[0, :, 0:1] + dacc[1, :, 0:1] + 1.0
    dis = lax.rsqrt(deg)
    g0[...] = dis * x[...]
    disb[...] = jnp.broadcast_to(dis, disb.shape)


def _comb_body(p, g0, db, g1):
    d = db[...]
    g1[...] = d * d * (p[0] + p[1] + g0[...])


def _final_body(q, g1, db, w, b, o):
    h = db[...] * (q[0] + q[1] + g1[...])
    o[...] = lax.dot_general(h, w[...], (((1,), (1,)), ((), ())),
                             preferred_element_type=jnp.float32) + b[...]


def _prep(dacc, xp):
    np_, d = xp.shape
    bn = np_ // TC_GRID
    bs = lambda shp: pl.BlockSpec(shp, lambda i: (i, 0))
    return pl.pallas_call(
        _prep_body,
        grid=(TC_GRID,),
        in_specs=[pl.BlockSpec((2, bn, d), lambda i: (0, i, 0)),
                  bs((bn, d))],
        out_specs=[bs((bn, d)), bs((bn, d))],
        out_shape=[jax.ShapeDtypeStruct((np_, d), jnp.float32)] * 2,
    )(dacc, xp)


def _comb(p, g0, disb):
    np_, d = g0.shape
    bn = np_ // TC_GRID
    bs = pl.BlockSpec((bn, d), lambda i: (i, 0))
    return pl.pallas_call(
        _comb_body,
        grid=(TC_GRID,),
        in_specs=[pl.BlockSpec((2, bn, d), lambda i: (0, i, 0)), bs, bs],
        out_specs=bs,
        out_shape=jax.ShapeDtypeStruct((np_, d), jnp.float32),
    )(p, g0, disb)


def _final(q, g1, disb, w, b2, n):
    np_, d = g1.shape
    bn = np_ // TC_GRID
    bs = pl.BlockSpec((bn, d), lambda i: (i, 0))
    return pl.pallas_call(
        _final_body,
        grid=(pl.cdiv(n, bn),),
        in_specs=[pl.BlockSpec((2, bn, d), lambda i: (0, i, 0)), bs, bs,
                  pl.BlockSpec((d, d), lambda i: (0, 0)),
                  pl.BlockSpec((1, d), lambda i: (0, 0))],
        out_specs=bs,
        out_shape=jax.ShapeDtypeStruct((n, d), jnp.float32),
    )(q, g1, disb, w, b2)


def kernel(x, edge_index, W, b):
    n, d = x.shape
    e = edge_index.shape[1]
    np_ = _round_up(n, NS * B)
    ep = _round_up(e, NW * B * 2)
    nblk = ep // (NW * B)

    rows = edge_index[0].astype(jnp.int32)
    cols = edge_index[1].astype(jnp.int32)
    pad = ep - e
    ar = jnp.arange(pad, dtype=jnp.int32)
    rows_p = jnp.concatenate([rows, ar % 64]).reshape(NW, nblk, B)
    cols_p = jnp.concatenate([cols, n + ar % (np_ - n)]).reshape(NW, nblk, B)
    xp = jnp.pad(x, ((0, np_ - n), (0, 0)))

    zd = jnp.zeros((np_, d), jnp.float32)
    ones_hbm = jnp.ones((B, d), jnp.float32)
    dacc = _deg_call(np_, nblk, d)(cols_p, zd, ones_hbm)
    g0, disb = _prep(dacc, xp)
    p = _hop_call(np_, nblk, d)(g0, rows_p, cols_p, zd)
    g1 = _comb(p, g0, disb)
    q = _hop_call(np_, nblk, d)(g1, rows_p, cols_p, zd)
    return _final(q, g1, disb, W, b.reshape(1, d), n)


# speedup vs baseline: 25.4082x; 1.1568x over previous
import functools

import jax
import jax.numpy as jnp
from jax import lax
from jax.experimental import pallas as pl
from jax.experimental.pallas import tpu as pltpu
from jax.experimental.pallas import tpu_sc as plsc

NC = 2
NS = 16
NW = NC * NS
B = 128
LANES = 16
TC_GRID = 16
DEG_K = 8


def _round_up(v, m):
    return (v + m - 1) // m * m


def _deg_body(np_, nblk, cols, zh, oh, dacc_out, ia, ones_b, acc):
    c = lax.axis_index("c")
    s = lax.axis_index("s")
    pltpu.sync_copy(oh, ones_b)

    rpt = np_ // NS
    pltpu.sync_copy(zh.at[pl.ds(s * rpt, rpt)], acc.at[pl.ds(s * rpt, rpt)])
    plsc.subcore_barrier()

    w = c * NS + s

    @pl.loop(0, nblk)
    def _edges(blk):
        pltpu.sync_copy(cols.at[w, blk], ia)
        pltpu.sync_copy(ones_b, acc.at[ia], add=True)

    plsc.subcore_barrier()
    pltpu.sync_copy(acc.at[pl.ds(s * rpt, rpt)],
                    dacc_out.at[c, pl.ds(s * rpt, rpt)])


def _hop_body(np_, nblk, d, g, rows, cols, zh, part_out,
              idx_r, idx_c, buf0, buf1, sem0, sem1, semir, semic, acc):
    c = lax.axis_index("c")
    s = lax.axis_index("s")

    rpt = np_ // NS
    pltpu.sync_copy(zh.at[pl.ds(s * rpt, rpt)], acc.at[pl.ds(s * rpt, rpt)])
    plsc.subcore_barrier()

    w = c * NS + s
    pltpu.sync_copy(rows.at[w, pl.ds(0, 2)], idx_r.at[0])
    pltpu.sync_copy(cols.at[w, pl.ds(0, 2)], idx_c.at[0])
    pltpu.async_copy(g.at[idx_r.at[0, 0]], buf0, sem0)
    pltpu.async_copy(rows.at[w, pl.ds(2, 2)], idx_r.at[1], semir)
    pltpu.async_copy(cols.at[w, pl.ds(2, 2)], idx_c.at[1], semic)

    def _pair(pr, p, q):
        pltpu.make_async_copy(g.at[idx_r.at[p, 0]], buf0, sem0).wait()
        pltpu.async_copy(g.at[idx_r.at[p, 1]], buf1, sem1)
        pltpu.sync_copy(buf0, acc.at[idx_c.at[p, 0]], add=True)
        pltpu.make_async_copy(g.at[idx_r.at[p, 1]], buf1, sem1).wait()
        pltpu.make_async_copy(rows.at[w, pl.ds(0, 2)], idx_r.at[q],
                              semir).wait()
        pltpu.make_async_copy(cols.at[w, pl.ds(0, 2)], idx_c.at[q],
                              semic).wait()
        pltpu.async_copy(g.at[idx_r.at[q, 0]], buf0, sem0)
        pltpu.sync_copy(buf1, acc.at[idx_c.at[p, 1]], add=True)
        nx = lax.rem(2 * pr + 4, nblk)
        pltpu.async_copy(rows.at[w, pl.ds(nx, 2)], idx_r.at[p], semir)
        pltpu.async_copy(cols.at[w, pl.ds(nx, 2)], idx_c.at[p], semic)

    @pl.loop(0, nblk // 2, step=2)
    def _edges(pr):
        _pair(pr, 0, 1)
        _pair(pr + 1, 1, 0)

    pltpu.make_async_copy(g.at[idx_r.at[0, 0]], buf0, sem0).wait()
    pltpu.make_async_copy(rows.at[w, pl.ds(0, 2)], idx_r.at[0], semir).wait()
    pltpu.make_async_copy(cols.at[w, pl.ds(0, 2)], idx_c.at[0], semic).wait()
    plsc.subcore_barrier()
    pltpu.sync_copy(acc.at[pl.ds(s * rpt, rpt)],
                    part_out.at[c, pl.ds(s * rpt, rpt), :])


@functools.lru_cache(maxsize=None)
def _deg_call(np_, nblk):
    mesh = plsc.VectorSubcoreMesh(core_axis_name="c", subcore_axis_name="s")
    return pl.kernel(
        functools.partial(_deg_body, np_, nblk),
        out_type=jax.ShapeDtypeStruct((NC, np_), jnp.float32),
        mesh=mesh,
        scratch_types=[
            pltpu.VMEM((B,), jnp.int32),
            pltpu.VMEM((B,), jnp.float32),
            pltpu.VMEM_SHARED((np_,), jnp.float32),
        ],
    )


@functools.lru_cache(maxsize=None)
def _hop_call(np_, nblk, d):
    mesh = plsc.VectorSubcoreMesh(core_axis_name="c", subcore_axis_name="s")
    return pl.kernel(
        functools.partial(_hop_body, np_, nblk, d),
        out_type=jax.ShapeDtypeStruct((NC, np_, d), jnp.float32),
        mesh=mesh,
        scratch_types=[
            pltpu.VMEM((2, 2, B), jnp.int32),
            pltpu.VMEM((2, 2, B), jnp.int32),
            pltpu.VMEM((B, d), jnp.float32),
            pltpu.VMEM((B, d), jnp.float32),
            pltpu.SemaphoreType.DMA,
            pltpu.SemaphoreType.DMA,
            pltpu.SemaphoreType.DMA,
            pltpu.SemaphoreType.DMA,
            pltpu.VMEM_SHARED((np_, d), jnp.float32),
        ],
    )


def _prep_body(dacc, x, g0, disb):
    dd = dacc[...]
    deg = (dd[0] + dd[1] + 1.0)[:, None]
    dis = lax.rsqrt(deg)
    g0[...] = dis * x[...]
    disb[...] = jnp.broadcast_to(dis, disb.shape)


def _comb_body(p, g0, db, g1):
    d = db[...]
    g1[...] = d * d * (p[0] + p[1] + g0[...])


def _final_body(q, g1, db, w, b, o):
    h = db[...] * (q[0] + q[1] + g1[...])
    o[...] = lax.dot_general(h, w[...], (((1,), (1,)), ((), ())),
                             preferred_element_type=jnp.float32) + b[...]


def _prep(dacc, xp):
    np_, d = xp.shape
    bn = np_ // TC_GRID
    bs = lambda shp: pl.BlockSpec(shp, lambda i: (i, 0))
    return pl.pallas_call(
        _prep_body,
        grid=(TC_GRID,),
        in_specs=[pl.BlockSpec((2, bn), lambda i: (0, i)),
                  bs((bn, d))],
        out_specs=[bs((bn, d)), bs((bn, d))],
        out_shape=[jax.ShapeDtypeStruct((np_, d), jnp.float32)] * 2,
    )(dacc, xp)


def _comb(p, g0, disb):
    np_, d = g0.shape
    bn = np_ // TC_GRID
    bs = pl.BlockSpec((bn, d), lambda i: (i, 0))
    return pl.pallas_call(
        _comb_body,
        grid=(TC_GRID,),
        in_specs=[pl.BlockSpec((2, bn, d), lambda i: (0, i, 0)), bs, bs],
        out_specs=bs,
        out_shape=jax.ShapeDtypeStruct((np_, d), jnp.float32),
    )(p, g0, disb)


def _final(q, g1, disb, w, b2, n):
    np_, d = g1.shape
    bn = np_ // TC_GRID
    bs = pl.BlockSpec((bn, d), lambda i: (i, 0))
    return pl.pallas_call(
        _final_body,
        grid=(pl.cdiv(n, bn),),
        in_specs=[pl.BlockSpec((2, bn, d), lambda i: (0, i, 0)), bs, bs,
                  pl.BlockSpec((d, d), lambda i: (0, 0)),
                  pl.BlockSpec((1, d), lambda i: (0, 0))],
        out_specs=bs,
        out_shape=jax.ShapeDtypeStruct((n, d), jnp.float32),
    )(q, g1, disb, w, b2)


def kernel(x, edge_index, W, b):
    n, d = x.shape
    e = edge_index.shape[1]
    np_ = _round_up(n, NS * B)
    ep = _round_up(e, NW * B * 2)
    nblk = ep // (NW * B)

    rows = edge_index[0].astype(jnp.int32)
    cols = edge_index[1].astype(jnp.int32)
    pad = ep - e
    ar = jnp.arange(pad, dtype=jnp.int32)
    rows_p = jnp.concatenate([rows, ar % 64]).reshape(NW, nblk, B)
    cols_p = jnp.concatenate([cols, n + ar % (np_ - n)]).reshape(NW, nblk, B)
    xp = jnp.pad(x, ((0, np_ - n), (0, 0)))

    zd = jnp.zeros((np_, d), jnp.float32)
    zdeg = jnp.zeros((np_,), jnp.float32)
    ones_hbm = jnp.ones((B,), jnp.float32)
    dacc = _deg_call(np_, nblk)(cols_p, zdeg, ones_hbm)
    g0, disb = _prep(dacc, xp)
    p = _hop_call(np_, nblk, d)(g0, rows_p, cols_p, zd)
    g1 = _comb(p, g0, disb)
    q = _hop_call(np_, nblk, d)(g1, rows_p, cols_p, zd)
    return _final(q, g1, disb, W, b.reshape(1, d), n)
